# Initial kernel scaffold; baseline (speedup 1.0000x reference)
import jax
import jax.numpy as jnp
from jax.experimental import pallas as pl


def kernel(x, edge_index, edge_attr, ptr, nnodes, params):
    raise NotImplementedError("write your pallas kernel here")


# speedup vs baseline: 9.6650x; 9.6650x over previous
import jax
import jax.numpy as jnp
from jax.experimental import pallas as pl
from jax.experimental.pallas import tpu as pltpu

_GNN_L = 3


def _tc_kernel(x_ref, eattr_ref, sl_ref, dl_ref,
               atom_w_ref, bond_w_ref, lw_ref, mlp_w1_ref, bias_ref,
               out_ref):
    NPG = x_ref.shape[0]
    EPG = eattr_ref.shape[0]
    HID = x_ref.shape[1]

    bp = bias_ref[...]
    f32 = jnp.float32

    def mm(a, b):
        return jax.lax.dot_general(
            a, b, (((1,), (0,)), ((), ())), preferred_element_type=f32)

    sl = sl_ref[0]
    dl = dl_ref[0]

    node_iota = jax.lax.broadcasted_iota(jnp.int32, (NPG, EPG), 0)
    oh_src_t = (node_iota == jnp.broadcast_to(sl, (NPG, EPG))).astype(f32)
    oh_dst_t = (node_iota == jnp.broadcast_to(dl, (NPG, EPG))).astype(f32)

    h = mm(x_ref[...], atom_w_ref[...]) + bp[0]
    ea = mm(eattr_ref[...], bond_w_ref[...]) + bp[1]

    for i in range(_GNN_L):
        w = lw_ref[i]
        b = bp[2 + 7 * i: 2 + 7 * (i + 1)]
        e = jax.nn.relu(mm(ea, w[0]) + b[0])
        e = mm(e, w[1]) + b[1]
        h_src = jax.lax.dot_general(
            oh_src_t, h, (((0,), (0,)), ((), ())), preferred_element_type=f32)
        m = jax.nn.relu(h_src + e)
        agg = mm(oh_dst_t, m)
        z = b[6] * h + agg
        z = jax.nn.relu(mm(z, w[2]) + b[2])
        z = mm(z, w[3]) + b[3]
        z = z * b[5] + b[4]
        h = jax.nn.relu(z)

    w1 = mlp_w1_ref[...]
    A = mm(h, w1[:HID])
    Bm = mm(h, w1[HID:2 * HID])
    P = mm(ea, w1[2 * HID:])

    pid = sl * NPG + dl
    NP2 = NPG * NPG
    pair_iota = jax.lax.broadcasted_iota(jnp.int32, (NP2, EPG), 0)
    poh = (pair_iota == jnp.broadcast_to(pid, (NP2, EPG))).astype(f32)
    S = mm(poh, P)

    a_rep = jnp.broadcast_to(A[:, None, :], (NPG, NPG, HID)).reshape(NP2, HID)
    b_tile = jnp.broadcast_to(Bm[None, :, :], (NPG, NPG, HID)).reshape(NP2, HID)
    q = jax.nn.relu(a_rep + b_tile + S + bp[23])
    out = jnp.sum(q * bp[24], axis=1, keepdims=True) + bp[25][0]
    out_ref[...] = out


def kernel(x, edge_index, edge_attr, ptr, nnodes, params):
    B = nnodes.shape[0]
    N, IN_F = x.shape
    NPG = N // B
    E = edge_index.shape[1]
    EPG = E // B
    HID = params['atom_W'].shape[1]
    NP2 = NPG * NPG

    src = edge_index[0].astype(jnp.int32)
    dst = edge_index[1].astype(jnp.int32)
    sl = jnp.reshape(src % NPG, (B, 1, EPG))
    dl = jnp.reshape(dst % NPG, (B, 1, EPG))

    lw = jnp.stack([
        jnp.stack([params[f'g{i}_be_W1'], params[f'g{i}_be_W2'],
                   params[f'g{i}_nn_W1'], params[f'g{i}_nn_W2']])
        for i in range(_GNN_L)])

    ones = jnp.ones((HID,), jnp.float32)
    rows = [params['atom_b'], params['bond_b']]
    bn_inv = 1.0 / jnp.sqrt(jnp.float32(1.0 + 1e-5))
    for i in range(_GNN_L):
        rows += [params[f'g{i}_be_b1'], params[f'g{i}_be_b2'],
                 params[f'g{i}_nn_b1'], params[f'g{i}_nn_b2'],
                 params[f'g{i}_bn_b'], params[f'g{i}_bn_g'] * bn_inv,
                 (1.0 + params[f'g{i}_eps']) * ones]
    rows += [params['mlp_b1'], params['mlp_W2'][:, 0], params['mlp_b2'][0] * ones]
    while len(rows) < 32:
        rows.append(jnp.zeros((HID,), jnp.float32))
    bias = jnp.stack(rows)

    grid = (B,)
    out = pl.pallas_call(
        _tc_kernel,
        grid=grid,
        in_specs=[
            pl.BlockSpec((NPG, IN_F), lambda g: (g, 0)),
            pl.BlockSpec((EPG, edge_attr.shape[1]), lambda g: (g, 0)),
            pl.BlockSpec((1, 1, EPG), lambda g: (g, 0, 0)),
            pl.BlockSpec((1, 1, EPG), lambda g: (g, 0, 0)),
            pl.BlockSpec((IN_F, HID), lambda g: (0, 0)),
            pl.BlockSpec((edge_attr.shape[1], HID), lambda g: (0, 0)),
            pl.BlockSpec((_GNN_L, 4, HID, HID), lambda g: (0, 0, 0, 0)),
            pl.BlockSpec((3 * HID, HID), lambda g: (0, 0)),
            pl.BlockSpec((32, HID), lambda g: (0, 0)),
        ],
        out_specs=pl.BlockSpec((NP2, 1), lambda g: (g, 0)),
        out_shape=jax.ShapeDtypeStruct((B * NP2, 1), jnp.float32),
        compiler_params=pltpu.CompilerParams(
            dimension_semantics=("parallel",)),
    )(x, edge_attr, sl, dl, params['atom_W'], params['bond_W'],
      lw, params['mlp_W1'], bias)
    return out


# speedup vs baseline: 24.6249x; 2.5478x over previous
import jax
import jax.numpy as jnp
from jax.experimental import pallas as pl
from jax.experimental.pallas import tpu as pltpu

_GNN_L = 3
_BN_INV = float(1.0 / (1.0 + 1e-5) ** 0.5)


def _tc_kernel(x_ref, eattr_ref, src_ref, dst_ref, pid_ref,
               atom_w_ref, bond_w_ref,
               w00, w01, w02, w03, w10, w11, w12, w13, w20, w21, w22, w23,
               mlp_w1_ref, w2t_ref,
               atom_b_ref, bond_b_ref,
               b00, b01, b02, b03, b04, b05, b06,
               b10, b11, b12, b13, b14, b15, b16,
               b20, b21, b22, b23, b24, b25, b26,
               mlp_b1_ref, mlp_b2_ref,
               out_ref):
    f32 = jnp.float32
    N, HID = x_ref.shape[0], atom_w_ref.shape[1]
    E = eattr_ref.shape[0]
    B = pid_ref.shape[0]
    NPG = N // B
    EPG = E // B
    NP2 = NPG * NPG
    GB = 4
    NB = GB * NPG
    EB = GB * EPG
    NBLK = B // GB

    def mm(a, b):
        return jax.lax.dot_general(
            a, b, (((1,), (0,)), ((), ())), preferred_element_type=f32)

    def mm_t(a, b):
        return jax.lax.dot_general(
            a, b, (((0,), (0,)), ((), ())), preferred_element_type=f32)

    lw = [[w00, w01, w02, w03], [w10, w11, w12, w13], [w20, w21, w22, w23]]
    lb = [[b00, b01, b02, b03, b04, b05, b06],
          [b10, b11, b12, b13, b14, b15, b16],
          [b20, b21, b22, b23, b24, b25, b26]]

    src = src_ref[...]
    dst = dst_ref[...]

    blk_iota = jax.lax.broadcasted_iota(jnp.int32, (NB, EB), 0)
    oh_src_t = []
    oh_dst_t = []
    for k in range(NBLK):
        s = jnp.broadcast_to(src[:, k * EB:(k + 1) * EB] - k * NB, (NB, EB))
        d = jnp.broadcast_to(dst[:, k * EB:(k + 1) * EB] - k * NB, (NB, EB))
        oh_src_t.append((blk_iota == s).astype(f32))
        oh_dst_t.append((blk_iota == d).astype(f32))

    h = mm(x_ref[...], atom_w_ref[...]) + atom_b_ref[...]
    ea = mm(eattr_ref[...], bond_w_ref[...]) + bond_b_ref[...]

    for i in range(_GNN_L):
        w, b = lw[i], lb[i]
        e = jax.nn.relu(mm(ea, w[0][...]) + b[0][...])
        e = mm(e, w[1][...]) + b[1][...]
        parts = []
        for k in range(NBLK):
            h_k = h[k * NB:(k + 1) * NB]
            h_src = mm_t(oh_src_t[k], h_k)
            m = jax.nn.relu(h_src + e[k * EB:(k + 1) * EB])
            parts.append(mm(oh_dst_t[k], m))
        agg = jnp.concatenate(parts, axis=0)
        eps1 = b[6][0, 0]
        z = eps1 * h + agg
        z = jax.nn.relu(mm(z, w[2][...]) + b[2][...])
        z = mm(z, w[3][...]) + b[3][...]
        z = z * (b[5][...] * _BN_INV) + b[4][...]
        h = jax.nn.relu(z)

    w1 = mlp_w1_ref[...]
    A = mm(h, w1[:HID]) + mlp_b1_ref[...]
    Bm = mm(h, w1[HID:2 * HID])
    P = mm(ea, w1[2 * HID:])

    w2t = w2t_ref[...]
    b2 = mlp_b2_ref[0, 0]
    pair_iota = jax.lax.broadcasted_iota(jnp.int32, (NP2, EPG), 0)
    for g in range(B):
        pid = pid_ref[g:g + 1]
        poh = (pair_iota == jnp.broadcast_to(pid, (NP2, EPG))).astype(f32)
        S = mm(poh, P[g * EPG:(g + 1) * EPG])
        A_g = A[g * NPG:(g + 1) * NPG]
        B_g = Bm[g * NPG:(g + 1) * NPG]
        a_rep = jnp.broadcast_to(
            A_g[:, None, :], (NPG, NPG, HID)).reshape(NP2, HID)
        b_tile = jnp.broadcast_to(
            B_g[None, :, :], (NPG, NPG, HID)).reshape(NP2, HID)
        q = jax.nn.relu(a_rep + b_tile + S)
        out_ref[g * NP2:(g + 1) * NP2, :] = (
            jnp.sum(q * w2t, axis=1, keepdims=True) + b2)


def kernel(x, edge_index, edge_attr, ptr, nnodes, params):
    B = nnodes.shape[0]
    N = x.shape[0]
    NPG = N // B
    E = edge_index.shape[1]
    EPG = E // B
    NP2 = NPG * NPG

    src = edge_index[0].astype(jnp.int32)
    dst = edge_index[1].astype(jnp.int32)
    pid = jnp.reshape((src % NPG) * NPG + (dst % NPG), (B, EPG))

    def row(v):
        return jnp.reshape(v, (1, -1))

    args = [x, edge_attr, jnp.reshape(src, (1, E)), jnp.reshape(dst, (1, E)),
            pid, params['atom_W'], params['bond_W']]
    for i in range(_GNN_L):
        args += [params[f'g{i}_be_W1'], params[f'g{i}_be_W2'],
                 params[f'g{i}_nn_W1'], params[f'g{i}_nn_W2']]
    args += [params['mlp_W1'], jnp.reshape(params['mlp_W2'], (1, -1))]
    args += [row(params['atom_b']), row(params['bond_b'])]
    for i in range(_GNN_L):
        args += [row(params[f'g{i}_be_b1']), row(params[f'g{i}_be_b2']),
                 row(params[f'g{i}_nn_b1']), row(params[f'g{i}_nn_b2']),
                 row(params[f'g{i}_bn_b']), row(params[f'g{i}_bn_g']),
                 jnp.reshape(1.0 + params[f'g{i}_eps'], (1, 1))]
    args += [row(params['mlp_b1']), jnp.reshape(params['mlp_b2'], (1, 1))]

    return pl.pallas_call(
        _tc_kernel,
        out_shape=jax.ShapeDtypeStruct((B * NP2, 1), jnp.float32),
    )(*args)
